# baseline (speedup 1.0000x reference)
import functools
import math

import jax
import jax.numpy as jnp
from jax import lax
from jax.experimental import pallas as pl
from jax.experimental.pallas import tpu as pltpu
from jax.experimental.pallas import tpu_sc as plsc

BINS = 64
VMIN = 0.0
VMAX = 2.0
SIGMA = 0.05
NCLS = 32
DELTA = (VMAX - VMIN) / BINS
KNORM = DELTA / (SIGMA * math.sqrt(2.0 * math.pi))
HALF = 7
TAPS = 16
RT_HALF = math.sqrt(0.5)
STEP = DELTA / SIGMA * RT_HALF

NBATCH = 4
NATOMS = 4096
NBONDS = 32768
NW = 32
BPW = NBONDS // NW
WPB = NW // NBATCH
HSZ = BINS * NCLS
L = 16


def _sc_body(atoms_hbm, idx_hbm, out_hbm, atoms_v, idx_v, hist_v):
    wid = lax.axis_index("s") * 2 + lax.axis_index("c")
    batch = wid // WPB

    pltpu.sync_copy(atoms_hbm.at[batch], atoms_v)
    pltpu.sync_copy(idx_hbm.at[pl.ds(wid * (3 * BPW), 3 * BPW)], idx_v)

    zeros = jnp.zeros((L,), jnp.float32)
    for zb in range(BINS):
        hist_v[zb, pl.ds(0, L)] = zeros
        hist_v[zb, pl.ds(L, L)] = zeros

    lane3 = jax.lax.iota(jnp.int32, L) * 3
    half_f = jnp.float32(0.5)
    inv_delta = jnp.float32(1.0 / DELTA)
    scale = jnp.float32(RT_HALF / SIGMA)
    knorm_f = jnp.float32(KNORM)
    magic = jnp.int32(0x5F3759DF)

    def group_body(g, carry):
        qv = lane3 + g * (3 * L)
        cls = plsc.load_gather(idx_v, [qv])
        i1 = plsc.load_gather(idx_v, [qv + 1])
        i2 = plsc.load_gather(idx_v, [qv + 2])
        a1 = i1 * 3
        a2 = i2 * 3

        dx = plsc.load_gather(atoms_v, [a1]) - plsc.load_gather(atoms_v, [a2])
        dy = plsc.load_gather(atoms_v, [a1 + 1]) - plsc.load_gather(atoms_v, [a2 + 1])
        dz = plsc.load_gather(atoms_v, [a1 + 2]) - plsc.load_gather(atoms_v, [a2 + 2])
        d2 = dx * dx + dy * dy + dz * dz

        bits = lax.bitcast_convert_type(d2, jnp.int32)
        bits = magic - lax.shift_right_arithmetic(bits, 1)
        y = lax.bitcast_convert_type(bits, jnp.float32)
        for _ in range(3):
            t = (d2 * y) * y
            y = y * (jnp.float32(1.5) - half_f * t)
        dis = d2 * y

        b0 = (dis * inv_delta).astype(jnp.int32)
        b0f = b0.astype(jnp.float32)
        v0 = dis * scale - (b0f + half_f) * jnp.float32(DELTA * RT_HALF / SIGMA)

        for tp in range(-HALF, TAPS - HALF):
            v = v0 - jnp.float32(tp * STEP)
            e = v * v
            w = jnp.exp(jnp.float32(0.0) - e) * knorm_f
            binv = b0 + jnp.int32(tp)
            if tp < 0:
                m = b0 >= jnp.int32(-tp)
                plsc.addupdate_scatter(hist_v, [binv, cls], w, mask=m)
            else:
                plsc.addupdate_scatter(hist_v, [binv, cls], w)
        return carry

    lax.fori_loop(0, BPW // L, group_body, 0)

    pltpu.sync_copy(hist_v, out_hbm.at[wid])


@jax.jit
def _sc_hist(atoms_r, idx_r):
    mesh = plsc.VectorSubcoreMesh(core_axis_name="c", subcore_axis_name="s")
    f = functools.partial(
        pl.kernel,
        mesh=mesh,
        out_type=jax.ShapeDtypeStruct((NW, BINS, NCLS), jnp.float32),
        scratch_types=[
            pltpu.VMEM((3 * NATOMS,), jnp.float32),
            pltpu.VMEM((3 * BPW,), jnp.int32),
            pltpu.VMEM((BINS, NCLS), jnp.float32),
        ],
        compiler_params=pltpu.CompilerParams(needs_layout_passes=False),
    )(_sc_body)
    return f(atoms_r, idx_r)


def _finish_body(p_ref, o_ref):
    x = p_ref[...]
    h = jnp.sum(x, axis=0) + jnp.float32(1e-40)
    s = jnp.sum(h, axis=0, keepdims=True)
    r = h / (s + jnp.float32(1e-20))
    o_ref[...] = r[:, : NCLS - 1]


@jax.jit
def _finish(partials):
    return pl.pallas_call(
        _finish_body,
        out_shape=jax.ShapeDtypeStruct((BINS, NCLS - 1), jnp.float32),
    )(partials)


def kernel(atoms, indices):
    atoms_r = atoms.reshape(NBATCH, 3 * NATOMS)
    idx_r = indices.astype(jnp.int32).reshape(3 * NBONDS)
    return _finish(_sc_hist(atoms_r, idx_r))


# speedup vs baseline: 2.0672x; 2.0672x over previous
import functools
import math

import jax
import jax.numpy as jnp
from jax import lax
from jax.experimental import pallas as pl
from jax.experimental.pallas import tpu as pltpu
from jax.experimental.pallas import tpu_sc as plsc

BINS = 64
VMIN = 0.0
VMAX = 2.0
SIGMA = 0.05
NCLS = 32
DELTA = (VMAX - VMIN) / BINS
KNORM = DELTA / (SIGMA * math.sqrt(2.0 * math.pi))
HALF = 7
TAPS = 16
RT_HALF = math.sqrt(0.5)
STEP = DELTA / SIGMA * RT_HALF

NBATCH = 4
NATOMS = 4096
NBONDS = 32768
NW = 32
BPW = NBONDS // NW
WPB = NW // NBATCH
HSZ = BINS * NCLS
L = 16


def _sc_body(atoms_hbm, i1_hbm, i2_hbm, cl_hbm, out_hbm,
             atoms_v, i1_v, i2_v, cl_v, hist_v):
    wid = lax.axis_index("s") * 2 + lax.axis_index("c")
    batch = wid // WPB
    base = wid * BPW

    pltpu.sync_copy(atoms_hbm.at[batch], atoms_v)
    pltpu.sync_copy(i1_hbm.at[pl.ds(base, BPW)], i1_v)
    pltpu.sync_copy(i2_hbm.at[pl.ds(base, BPW)], i2_v)
    pltpu.sync_copy(cl_hbm.at[pl.ds(base, BPW)], cl_v)

    zeros = jnp.zeros((L,), jnp.float32)
    for zb in range(BINS):
        hist_v[zb, pl.ds(0, L)] = zeros
        hist_v[zb, pl.ds(L, L)] = zeros

    half_f = jnp.float32(0.5)
    inv_delta = jnp.float32(1.0 / DELTA)
    scale = jnp.float32(RT_HALF / SIGMA)
    knorm_f = jnp.float32(KNORM)
    magic = jnp.int32(0x5F3759DF)

    def group_body(g, carry):
        off = g * L
        i1 = i1_v[pl.ds(off, L)]
        i2 = i2_v[pl.ds(off, L)]
        cls = cl_v[pl.ds(off, L)]

        dx = plsc.load_gather(atoms_v, [i1]) - plsc.load_gather(atoms_v, [i2])
        dy = plsc.load_gather(atoms_v, [i1 + NATOMS]) - plsc.load_gather(atoms_v, [i2 + NATOMS])
        dz = plsc.load_gather(atoms_v, [i1 + 2 * NATOMS]) - plsc.load_gather(atoms_v, [i2 + 2 * NATOMS])
        d2 = dx * dx + dy * dy + dz * dz

        bits = lax.bitcast_convert_type(d2, jnp.int32)
        bits = magic - lax.shift_right_arithmetic(bits, 1)
        y = lax.bitcast_convert_type(bits, jnp.float32)
        for _ in range(3):
            t = (d2 * y) * y
            y = y * (jnp.float32(1.5) - half_f * t)
        dis = d2 * y

        b0 = (dis * inv_delta).astype(jnp.int32)
        b0f = b0.astype(jnp.float32)
        v0 = dis * scale - (b0f + half_f) * jnp.float32(DELTA * RT_HALF / SIGMA)

        for tp in range(-HALF, TAPS - HALF):
            v = v0 - jnp.float32(tp * STEP)
            e = v * v
            w = jnp.exp(jnp.float32(0.0) - e) * knorm_f
            binv = b0 + jnp.int32(tp)
            if tp < 0:
                m = b0 >= jnp.int32(-tp)
                plsc.addupdate_scatter(hist_v, [binv, cls], w, mask=m)
            else:
                plsc.addupdate_scatter(hist_v, [binv, cls], w)
        return carry

    lax.fori_loop(0, BPW // L, group_body, 0)

    pltpu.sync_copy(hist_v, out_hbm.at[wid])


@jax.jit
def _sc_hist(atoms_t, i1, i2, cl):
    mesh = plsc.VectorSubcoreMesh(core_axis_name="c", subcore_axis_name="s")
    f = functools.partial(
        pl.kernel,
        mesh=mesh,
        out_type=jax.ShapeDtypeStruct((NW, BINS, NCLS), jnp.float32),
        scratch_types=[
            pltpu.VMEM((3 * NATOMS,), jnp.float32),
            pltpu.VMEM((BPW,), jnp.int32),
            pltpu.VMEM((BPW,), jnp.int32),
            pltpu.VMEM((BPW,), jnp.int32),
            pltpu.VMEM((BINS, NCLS), jnp.float32),
        ],
        compiler_params=pltpu.CompilerParams(needs_layout_passes=False),
    )(_sc_body)
    return f(atoms_t, i1, i2, cl)


def _finish_body(p_ref, o_ref):
    x = p_ref[...]
    h = jnp.sum(x, axis=0) + jnp.float32(1e-40)
    s = jnp.sum(h, axis=0, keepdims=True)
    r = h / (s + jnp.float32(1e-20))
    o_ref[...] = r[:, : NCLS - 1]


@jax.jit
def _finish(partials):
    return pl.pallas_call(
        _finish_body,
        out_shape=jax.ShapeDtypeStruct((BINS, NCLS - 1), jnp.float32),
    )(partials)


def kernel(atoms, indices):
    idx = indices.astype(jnp.int32)
    atoms_t = atoms.transpose(0, 2, 1).reshape(NBATCH, 3 * NATOMS)
    i1 = idx[:, :, 1].reshape(-1)
    i2 = idx[:, :, 2].reshape(-1)
    cl = idx[:, :, 0].reshape(-1)
    return _finish(_sc_hist(atoms_t, i1, i2, cl))


# speedup vs baseline: 2.2277x; 1.0776x over previous
import functools
import math

import jax
import jax.numpy as jnp
from jax import lax
from jax.experimental import pallas as pl
from jax.experimental.pallas import tpu as pltpu
from jax.experimental.pallas import tpu_sc as plsc

BINS = 64
VMIN = 0.0
VMAX = 2.0
SIGMA = 0.05
NCLS = 32
DELTA = (VMAX - VMIN) / BINS
KNORM = DELTA / (SIGMA * math.sqrt(2.0 * math.pi))
HALF = 5
TAPS = 12
RT_HALF = math.sqrt(0.5)
STEP = DELTA * RT_HALF / SIGMA

NBATCH = 4
NATOMS = 4096
NBONDS = 32768
NW = 32
BPW = NBONDS // NW
WPB = NW // NBATCH
L = 16
UNROLL = 2


def _sc_body(atoms_hbm, pk_hbm, out_hbm, atoms_v, pk_v, hist_v):
    wid = lax.axis_index("s") * 2 + lax.axis_index("c")
    batch = wid // WPB
    base = wid * BPW

    pltpu.sync_copy(atoms_hbm.at[batch], atoms_v)
    pltpu.sync_copy(pk_hbm.at[pl.ds(base, BPW)], pk_v)

    zeros = jnp.zeros((L,), jnp.float32)
    for zb in range(BINS):
        hist_v[zb, pl.ds(0, L)] = zeros
        hist_v[zb, pl.ds(L, L)] = zeros

    half_f = jnp.float32(0.5)
    inv_delta = jnp.float32(1.0 / DELTA)
    scale = jnp.float32(RT_HALF / SIGMA)
    lnk_f = jnp.float32(math.log(KNORM))
    magic = jnp.int32(0x5F3759DF)
    m31 = jnp.int32(31)
    m4095 = jnp.int32(4095)

    def one_group(off):
        pk = pk_v[pl.ds(off, L)]
        cls = lax.bitwise_and(pk, m31)
        i1 = lax.bitwise_and(lax.shift_right_logical(pk, 5), m4095)
        i2 = lax.shift_right_logical(pk, 17)

        dx = plsc.load_gather(atoms_v, [i1]) - plsc.load_gather(atoms_v, [i2])
        dy = plsc.load_gather(atoms_v, [i1 + NATOMS]) - plsc.load_gather(atoms_v, [i2 + NATOMS])
        dz = plsc.load_gather(atoms_v, [i1 + 2 * NATOMS]) - plsc.load_gather(atoms_v, [i2 + 2 * NATOMS])
        d2 = dx * dx + dy * dy + dz * dz

        bits = lax.bitcast_convert_type(d2, jnp.int32)
        bits = magic - lax.shift_right_arithmetic(bits, 1)
        y = lax.bitcast_convert_type(bits, jnp.float32)
        for _ in range(2):
            t = (d2 * y) * y
            y = y * (jnp.float32(1.5) - half_f * t)
        dis = d2 * y

        b0 = (dis * inv_delta).astype(jnp.int32)
        b0f = b0.astype(jnp.float32)
        v0 = dis * scale - (b0f + half_f) * jnp.float32(STEP)

        for tp in range(-HALF, TAPS - HALF):
            v = v0 - jnp.float32(tp * STEP)
            e = v * v
            w = jnp.exp(lnk_f - e)
            binv = b0 + jnp.int32(tp)
            if tp < 0:
                m = b0 >= jnp.int32(-tp)
                plsc.addupdate_scatter(hist_v, [binv, cls], w, mask=m)
            else:
                plsc.addupdate_scatter(hist_v, [binv, cls], w)

    def group_body(g, carry):
        for u in range(UNROLL):
            one_group(g * (UNROLL * L) + u * L)
        return carry

    lax.fori_loop(0, BPW // (UNROLL * L), group_body, 0)

    pltpu.sync_copy(hist_v, out_hbm.at[wid])


@jax.jit
def _sc_hist(atoms_t, pk):
    mesh = plsc.VectorSubcoreMesh(core_axis_name="c", subcore_axis_name="s")
    f = functools.partial(
        pl.kernel,
        mesh=mesh,
        out_type=jax.ShapeDtypeStruct((NW, BINS, NCLS), jnp.float32),
        scratch_types=[
            pltpu.VMEM((3 * NATOMS,), jnp.float32),
            pltpu.VMEM((BPW,), jnp.int32),
            pltpu.VMEM((BINS, NCLS), jnp.float32),
        ],
        compiler_params=pltpu.CompilerParams(needs_layout_passes=False),
    )(_sc_body)
    return f(atoms_t, pk)


def _finish_body(p_ref, o_ref):
    x = p_ref[...]
    h = jnp.sum(x, axis=0) + jnp.float32(1e-40)
    s = jnp.sum(h, axis=0, keepdims=True)
    r = h / (s + jnp.float32(1e-20))
    o_ref[...] = r[:, : NCLS - 1]


@jax.jit
def _finish(partials):
    return pl.pallas_call(
        _finish_body,
        out_shape=jax.ShapeDtypeStruct((BINS, NCLS - 1), jnp.float32),
    )(partials)


def kernel(atoms, indices):
    idx = indices.astype(jnp.int32)
    atoms_t = atoms.transpose(0, 2, 1).reshape(NBATCH, 3 * NATOMS)
    pk = (idx[:, :, 0]
          | (idx[:, :, 1] << 5)
          | (idx[:, :, 2] << 17)).reshape(-1)
    return _finish(_sc_hist(atoms_t, pk))
